# baseline (speedup 1.0000x reference)
import functools

import jax
import jax.numpy as jnp
import numpy as np
from jax import lax
from jax.experimental import pallas as pl
from jax.experimental.pallas import tpu as pltpu
from jax.experimental.pallas import tpu_sc as plsc

B = 4096
L = 200
EMBED = 32
HID = 128
CLASSES = 10

_info = plsc.get_sparse_core_info()
NC = _info.num_cores
NS = _info.num_subcores
LANES = _info.num_lanes
NW = NC * NS
BPW = B // NW
CB = 2
NCHUNK = BPW // CB
ROWS = CB * L
G = 80
NG = ROWS // G


def _sc_pool(x2, w_word, w_bigram, w_field):
    mesh = plsc.VectorSubcoreMesh(core_axis_name="c", subcore_axis_name="s")

    @functools.partial(
        pl.kernel,
        mesh=mesh,
        out_type=jax.ShapeDtypeStruct((B * 2 * EMBED,), jnp.float32),
        scratch_types=[
            pltpu.VMEM((3 * ROWS,), jnp.int32),
            pltpu.VMEM((3 * ROWS,), jnp.int32),
            pltpu.VMEM((3 * ROWS, EMBED // 2), jnp.float32),
            pltpu.VMEM((3 * ROWS, EMBED // 2), jnp.float32),
            pltpu.VMEM((BPW * 2 * EMBED,), jnp.float32),
            pltpu.SemaphoreType.DMA,
            pltpu.SemaphoreType.DMA,
        ],
        compiler_params=pltpu.CompilerParams(
            use_tc_tiling_on_sc=False, needs_layout_passes=False),
    )
    def body(x2_hbm, ww_hbm, wb_hbm, wf_hbm, out_hbm,
             idx0, idx1, rows0, rows1, acc_v, sem0, sem1):
        wid = lax.axis_index("s") * NC + lax.axis_index("c")
        tabs = (ww_hbm, wb_hbm, wf_hbm)
        idxs = (idx0, idx1)
        rows = (rows0, rows1)
        sems = (sem0, sem1)

        def issue(c, slot):
            off0 = pl.multiple_of((wid * BPW + c * CB) * L, ROWS)
            for t in range(3):
                pltpu.sync_copy(
                    x2_hbm.at[pl.ds(t * (B * L) + off0, ROWS)],
                    idxs[slot].at[pl.ds(t * ROWS, ROWS)])
            for t in range(3):
                for j in range(NG):
                    o = t * ROWS + j * G
                    pltpu.async_copy(
                        tabs[t].at[idxs[slot].at[pl.ds(o, G)]],
                        rows[slot].at[pl.ds(o, G)],
                        sems[slot])

        def drain(slot):
            pltpu.make_async_copy(
                tabs[0].at[pl.ds(0, 3 * ROWS)], rows[slot], sems[slot]).wait()

        def accumulate(c, slot):
            rv = rows[slot]
            for b in range(CB):
                def acc_body(l, carry):
                    a0, a1, g0, g1, f0, f1 = carry
                    r = b * L + l
                    fmt = plsc.PackFormat.INTERLEAVED
                    we, wo = plsc.unpack(
                        plsc.bitcast(rv[r], jnp.bfloat16), format=fmt)
                    ge, go = plsc.unpack(
                        plsc.bitcast(rv[ROWS + r], jnp.bfloat16), format=fmt)
                    fe, fo = plsc.unpack(
                        plsc.bitcast(rv[2 * ROWS + r], jnp.bfloat16),
                        format=fmt)
                    return (a0 + we, a1 + wo, g0 + ge, g1 + go,
                            f0 + fe, f1 + fo)
                z = jnp.zeros((LANES,), jnp.float32)
                a0, a1, g0, g1, f0, f1 = lax.fori_loop(
                    0, L, acc_body, (z, z, z, z, z, z))
                inv = jnp.float32(1.0 / L)
                o = (c * CB + b) * 2 * EMBED
                acc_v[pl.ds(o, LANES)] = (a0 + 5.0 * f0) * inv
                acc_v[pl.ds(o + LANES, LANES)] = (a1 + 5.0 * f1) * inv
                acc_v[pl.ds(o + 2 * LANES, LANES)] = g0 * inv
                acc_v[pl.ds(o + 3 * LANES, LANES)] = g1 * inv

        issue(0, 0)

        def pair_body(i, _):
            c0 = i * 2
            issue(c0 + 1, 1)
            drain(0)
            accumulate(c0, 0)

            @pl.when(c0 + 2 < NCHUNK)
            def _():
                issue(c0 + 2, 0)
            drain(1)
            accumulate(c0 + 1, 1)
            return 0

        lax.fori_loop(0, NCHUNK // 2, pair_body, 0)
        pltpu.sync_copy(
            acc_v, out_hbm.at[pl.ds(wid * BPW * 2 * EMBED, BPW * 2 * EMBED)])

    return body(x2, w_word, w_bigram, w_field)


VC = 8192


_SEL_E = np.zeros((EMBED, EMBED // 2), np.float32)
_SEL_E[np.arange(0, EMBED, 2), np.arange(EMBED // 2)] = 1.0
_SEL_O = np.zeros((EMBED, EMBED // 2), np.float32)
_SEL_O[np.arange(1, EMBED, 2), np.arange(EMBED // 2)] = 1.0


def _repack_body(se_ref, so_ref, i0, i1, i2, o0, o1, o2):
    dn = (((0,), (0,)), ((), ()))
    se = se_ref[...]
    so = so_ref[...]
    for i_ref, o_ref in ((i0, o0), (i1, o1), (i2, o2)):
        x = i_ref[...]
        e = lax.dot_general(x, se, dn, preferred_element_type=jnp.float32)
        o = lax.dot_general(x, so, dn, preferred_element_type=jnp.float32)
        ue = lax.bitcast_convert_type(e.astype(jnp.bfloat16), jnp.uint16)
        uo = lax.bitcast_convert_type(o.astype(jnp.bfloat16), jnp.uint16)
        word = (uo.astype(jnp.uint32) << 16) | ue.astype(jnp.uint32)
        o_ref[...] = lax.bitcast_convert_type(word, jnp.float32)


def _repack(wt_word, wt_bigram, wt_field):
    v = wt_word.shape[1]
    grid = ((v + VC - 1) // VC,)
    sel_spec = pl.BlockSpec((EMBED, EMBED // 2), lambda i: (0, 0))
    in_spec = pl.BlockSpec((EMBED, VC), lambda i: (0, i))
    out_spec = pl.BlockSpec((VC, EMBED // 2), lambda i: (i, 0))
    shp = jax.ShapeDtypeStruct((v, EMBED // 2), jnp.float32)
    return pl.pallas_call(
        _repack_body,
        grid=grid,
        in_specs=[sel_spec] * 2 + [in_spec] * 3,
        out_specs=[out_spec] * 3,
        out_shape=[shp] * 3,
    )(jnp.asarray(_SEL_E), jnp.asarray(_SEL_O),
      wt_word, wt_bigram, wt_field)


def _mlp_body(p_ref, w1_ref, b1_ref, w2_ref, b2_ref, o_ref):
    h = jnp.dot(p_ref[...], w1_ref[...], preferred_element_type=jnp.float32)
    h = jnp.maximum(h + b1_ref[...], 0.0)
    o_ref[...] = (
        jnp.dot(h, w2_ref[...], preferred_element_type=jnp.float32)
        + b2_ref[...]
    )


_PERM = np.concatenate([
    np.arange(0, EMBED, 2), np.arange(1, EMBED, 2),
    EMBED + np.arange(0, EMBED, 2), EMBED + np.arange(1, EMBED, 2),
])


def _mlp(pooled, fc1_w, fc1_b, fc2_w, fc2_b):
    w1p = fc1_w[_PERM, :]
    w2p = jnp.pad(fc2_w, ((0, 0), (0, HID - CLASSES)))
    b2p = jnp.pad(fc2_b, (0, HID - CLASSES)).reshape(1, HID)
    out = pl.pallas_call(
        _mlp_body,
        out_shape=jax.ShapeDtypeStruct((B, HID), jnp.float32),
    )(pooled, w1p, fc1_b.reshape(1, HID), w2p, b2p)
    return out[:, :CLASSES]


def kernel(x, W_word, W_bigram, W_field, fc1_w, fc1_b, fc2_w, fc2_b):
    x2 = x.reshape(3 * B * L).astype(jnp.int32)
    ww, wb, wf = _repack(W_word.T, W_bigram.T, W_field.T)
    pooled = _sc_pool(x2, ww, wb, wf).reshape(B, 2 * EMBED)
    return _mlp(pooled, fc1_w, fc1_b, fc2_w, fc2_b)


# speedup vs baseline: 1.3802x; 1.3802x over previous
import functools

import jax
import jax.numpy as jnp
import numpy as np
from jax import lax
from jax.experimental import pallas as pl
from jax.experimental.pallas import tpu as pltpu
from jax.experimental.pallas import tpu_sc as plsc

B = 4096
L = 200
EMBED = 32
HID = 128
CLASSES = 10

_info = plsc.get_sparse_core_info()
NC = _info.num_cores
NS = _info.num_subcores
LANES = _info.num_lanes
NW = NC * NS
BPW = B // NW
CB = 2
NCHUNK = BPW // CB
ROWS = CB * L
G = 80
NG = ROWS // G


def _sc_pool(x2, w_word, w_bigram, w_field):
    mesh = plsc.VectorSubcoreMesh(core_axis_name="c", subcore_axis_name="s")

    @functools.partial(
        pl.kernel,
        mesh=mesh,
        out_type=jax.ShapeDtypeStruct((B * 2 * EMBED,), jnp.float32),
        scratch_types=[
            pltpu.VMEM((3 * ROWS,), jnp.int32),
            pltpu.VMEM((3 * ROWS,), jnp.int32),
            pltpu.VMEM((3 * ROWS, EMBED // 2), jnp.float32),
            pltpu.VMEM((3 * ROWS, EMBED // 2), jnp.float32),
            pltpu.VMEM((BPW * 2 * EMBED,), jnp.float32),
            pltpu.SemaphoreType.DMA,
            pltpu.SemaphoreType.DMA,
        ],
        compiler_params=pltpu.CompilerParams(
            use_tc_tiling_on_sc=False, needs_layout_passes=False),
    )
    def body(x2_hbm, ww_hbm, wb_hbm, wf_hbm, out_hbm,
             idx0, idx1, rows0, rows1, acc_v, sem0, sem1):
        wid = lax.axis_index("s") * NC + lax.axis_index("c")
        tabs = (ww_hbm, wb_hbm, wf_hbm)
        idxs = (idx0, idx1)
        rows = (rows0, rows1)
        sems = (sem0, sem1)

        def issue(c, slot):
            off0 = pl.multiple_of((wid * BPW + c * CB) * L, ROWS)
            for t in range(3):
                pltpu.sync_copy(
                    x2_hbm.at[pl.ds(t * (B * L) + off0, ROWS)],
                    idxs[slot].at[pl.ds(t * ROWS, ROWS)])
            for t in range(3):
                for j in range(NG):
                    o = t * ROWS + j * G
                    pltpu.async_copy(
                        tabs[t].at[idxs[slot].at[pl.ds(o, G)]],
                        rows[slot].at[pl.ds(o, G)],
                        sems[slot])

        def drain(slot):
            pltpu.make_async_copy(
                tabs[0].at[pl.ds(0, 3 * ROWS)], rows[slot], sems[slot]).wait()

        def accumulate(c, slot):
            rv = rows[slot]
            for b in range(CB):
                def acc_body(l, carry):
                    a0, a1, g0, g1, f0, f1 = carry
                    r = b * L + l
                    fmt = plsc.PackFormat.INTERLEAVED
                    we, wo = plsc.unpack(
                        plsc.bitcast(rv[r], jnp.bfloat16), format=fmt)
                    ge, go = plsc.unpack(
                        plsc.bitcast(rv[ROWS + r], jnp.bfloat16), format=fmt)
                    fe, fo = plsc.unpack(
                        plsc.bitcast(rv[2 * ROWS + r], jnp.bfloat16),
                        format=fmt)
                    return (a0 + we, a1 + wo, g0 + ge, g1 + go,
                            f0 + fe, f1 + fo)
                z = jnp.zeros((LANES,), jnp.float32)
                a0, a1, g0, g1, f0, f1 = lax.fori_loop(
                    0, L, acc_body, (z, z, z, z, z, z))
                inv = jnp.float32(1.0 / L)
                o = (c * CB + b) * 2 * EMBED
                acc_v[pl.ds(o, LANES)] = (a0 + 5.0 * f0) * inv
                acc_v[pl.ds(o + LANES, LANES)] = (a1 + 5.0 * f1) * inv
                acc_v[pl.ds(o + 2 * LANES, LANES)] = g0 * inv
                acc_v[pl.ds(o + 3 * LANES, LANES)] = g1 * inv

        issue(0, 0)

        def pair_body(i, _):
            c0 = i * 2
            issue(c0 + 1, 1)
            drain(0)
            accumulate(c0, 0)

            @pl.when(c0 + 2 < NCHUNK)
            def _():
                issue(c0 + 2, 0)
            drain(1)
            accumulate(c0 + 1, 1)
            return 0

        lax.fori_loop(0, NCHUNK // 2, pair_body, 0)
        pltpu.sync_copy(
            acc_v, out_hbm.at[pl.ds(wid * BPW * 2 * EMBED, BPW * 2 * EMBED)])

    return body(x2, w_word, w_bigram, w_field)


VC = 1024
VQ = 125000
SS = 124928
MAIN = 8 * SS
TS = 72


_SEL_E = np.zeros((EMBED, EMBED // 2), np.float32)
_SEL_E[np.arange(0, EMBED, 2), np.arange(EMBED // 2)] = 1.0
_SEL_O = np.zeros((EMBED, EMBED // 2), np.float32)
_SEL_O[np.arange(1, EMBED, 2), np.arange(EMBED // 2)] = 1.0


def _pack_words(x, se, so):
    dn = (((0,), (0,)), ((), ()))
    e = lax.dot_general(x, se, dn, preferred_element_type=jnp.float32)
    o = lax.dot_general(x, so, dn, preferred_element_type=jnp.float32)
    ue = lax.bitcast_convert_type(e.astype(jnp.bfloat16), jnp.uint16)
    uo = lax.bitcast_convert_type(o.astype(jnp.bfloat16), jnp.uint16)
    word = (uo.astype(jnp.uint32) << 16) | ue.astype(jnp.uint32)
    return lax.bitcast_convert_type(word, jnp.float32)


def _repack_body(se_ref, so_ref, *refs):
    se = se_ref[...]
    so = so_ref[...]
    ins, outs = refs[:24], refs[24:]
    for t in range(3):
        for s in range(8):
            outs[t][:, 16 * s:16 * (s + 1)] = _pack_words(
                ins[8 * t + s][...], se, so)


def _repack(wt_word, wt_bigram, wt_field):
    grid = (SS // VC,)
    sel_spec = pl.BlockSpec((EMBED, EMBED // 2), lambda i: (0, 0))
    in_specs = [sel_spec] * 2
    for _ in range(3):
        for s in range(8):
            off = s * (SS // VC)
            in_specs.append(pl.BlockSpec(
                (EMBED, VC), lambda i, off=off: (0, off + i)))
    out_spec = pl.BlockSpec((VC, 128), lambda i: (i, 0))
    shp = jax.ShapeDtypeStruct((SS, 128), jnp.float32)
    tabs = (wt_word, wt_bigram, wt_field)
    return pl.pallas_call(
        _repack_body,
        grid=grid,
        in_specs=in_specs,
        out_specs=[out_spec] * 3,
        out_shape=[shp] * 3,
        compiler_params=pltpu.CompilerParams(
            fuse_transposed_lhs_in_matmul=True),
    )(jnp.asarray(_SEL_E), jnp.asarray(_SEL_O),
      *[t for t in tabs for _ in range(8)])


def _repack_tail_body(se_ref, so_ref, i0, i1, i2, o0, o1, o2):
    se = se_ref[...]
    so = so_ref[...]
    for i_ref, o_ref in ((i0, o0), (i1, o1), (i2, o2)):
        x3 = i_ref[...]
        for s in range(8):
            o_ref[:, 16 * s:16 * (s + 1)] = _pack_words(x3[:, s, :], se, so)


def _repack_tail(wt_word, wt_bigram, wt_field):
    t3 = [w[:, MAIN:].reshape(EMBED, 8, TS)
          for w in (wt_word, wt_bigram, wt_field)]
    shp = jax.ShapeDtypeStruct((TS, 128), jnp.float32)
    return pl.pallas_call(
        _repack_tail_body,
        out_shape=[shp] * 3,
    )(jnp.asarray(_SEL_E), jnp.asarray(_SEL_O), *t3)


_CPR = 128
_NCH = SS // _CPR


def _sc_linearize(m0, m1, m2, t0, t1, t2):
    mesh = plsc.VectorSubcoreMesh(core_axis_name="c", subcore_axis_name="s")
    shp = jax.ShapeDtypeStruct((VQ, 128), jnp.float32)

    @functools.partial(
        pl.kernel,
        mesh=mesh,
        out_type=(shp, shp, shp),
        scratch_types=[pltpu.VMEM((_CPR, 128), jnp.float32)],
        compiler_params=pltpu.CompilerParams(
            use_tc_tiling_on_sc=False, needs_layout_passes=False),
    )
    def body(m0, m1, m2, t0, t1, t2, o0, o1, o2, vbuf):
        wid = lax.axis_index("s") * NC + lax.axis_index("c")
        mains = (m0, m1, m2)
        tails = (t0, t1, t2)
        outs = (o0, o1, o2)

        def chunk_body(k, _):
            cid = k * NW + wid

            @pl.when(cid < _NCH)
            def _():
                r0 = pl.multiple_of(cid * _CPR, _CPR)
                for src, dst in zip(mains, outs):
                    pltpu.sync_copy(src.at[pl.ds(r0, _CPR)], vbuf)
                    pltpu.sync_copy(vbuf, dst.at[pl.ds(r0, _CPR)])
            return 0

        lax.fori_loop(0, (_NCH + NW - 1) // NW, chunk_body, 0)

        @pl.when(wid == 0)
        def _():
            for src, dst in zip(tails, outs):
                pltpu.sync_copy(src, vbuf.at[pl.ds(0, TS)])
                pltpu.sync_copy(vbuf.at[pl.ds(0, TS)], dst.at[pl.ds(SS, TS)])

    return body(m0, m1, m2, t0, t1, t2)


def _mlp_body(p_ref, w1_ref, b1_ref, w2_ref, b2_ref, o_ref):
    h = jnp.dot(p_ref[...], w1_ref[...], preferred_element_type=jnp.float32)
    h = jnp.maximum(h + b1_ref[...], 0.0)
    o_ref[...] = (
        jnp.dot(h, w2_ref[...], preferred_element_type=jnp.float32)
        + b2_ref[...]
    )


_PERM = np.concatenate([
    np.arange(0, EMBED, 2), np.arange(1, EMBED, 2),
    EMBED + np.arange(0, EMBED, 2), EMBED + np.arange(1, EMBED, 2),
])


def _mlp(pooled, fc1_w, fc1_b, fc2_w, fc2_b):
    w1p = fc1_w[_PERM, :]
    w2p = jnp.pad(fc2_w, ((0, 0), (0, HID - CLASSES)))
    b2p = jnp.pad(fc2_b, (0, HID - CLASSES)).reshape(1, HID)
    out = pl.pallas_call(
        _mlp_body,
        out_shape=jax.ShapeDtypeStruct((B, HID), jnp.float32),
    )(pooled, w1p, fc1_b.reshape(1, HID), w2p, b2p)
    return out[:, :CLASSES]


def kernel(x, W_word, W_bigram, W_field, fc1_w, fc1_b, fc2_w, fc2_b):
    xi = x.astype(jnp.int32)
    xt = xi - MAIN
    xp = jnp.where(xi < MAIN,
                   8 * (xi % SS) + xi // SS,
                   MAIN + 8 * (xt % TS) + xt // TS)
    x2 = xp.reshape(3 * B * L)
    wts = (W_word.T, W_bigram.T, W_field.T)
    mm = _repack(*wts)
    tt = _repack_tail(*wts)
    ww, wb, wf = _sc_linearize(*mm, *tt)
    p16 = (VQ * 8, EMBED // 2)
    pooled = _sc_pool(x2, ww.reshape(p16), wb.reshape(p16),
                      wf.reshape(p16)).reshape(B, 2 * EMBED)
    return _mlp(pooled, fc1_w, fc1_b, fc2_w, fc2_b)


# speedup vs baseline: 1.4888x; 1.0787x over previous
import functools

import jax
import jax.numpy as jnp
import numpy as np
from jax import lax
from jax.experimental import pallas as pl
from jax.experimental.pallas import tpu as pltpu
from jax.experimental.pallas import tpu_sc as plsc

B = 4096
L = 200
EMBED = 32
HID = 128
CLASSES = 10

_info = plsc.get_sparse_core_info()
NC = _info.num_cores
NS = _info.num_subcores
LANES = _info.num_lanes
NW = NC * NS
BPW = B // NW
CB = 2
NCHUNK = BPW // CB
ROWS = CB * L
G = 80
NG = ROWS // G


def _sc_pool(x2, w_word, w_bigram, w_field):
    mesh = plsc.VectorSubcoreMesh(core_axis_name="c", subcore_axis_name="s")

    @functools.partial(
        pl.kernel,
        mesh=mesh,
        out_type=jax.ShapeDtypeStruct((B * 2 * EMBED,), jnp.float32),
        scratch_types=[
            pltpu.VMEM((3 * ROWS,), jnp.int32),
            pltpu.VMEM((3 * ROWS,), jnp.int32),
            pltpu.VMEM((3 * ROWS, EMBED // 2), jnp.float32),
            pltpu.VMEM((3 * ROWS, EMBED // 2), jnp.float32),
            pltpu.VMEM((BPW * 2 * EMBED,), jnp.float32),
            pltpu.SemaphoreType.DMA,
            pltpu.SemaphoreType.DMA,
        ],
        compiler_params=pltpu.CompilerParams(
            use_tc_tiling_on_sc=False, needs_layout_passes=False),
    )
    def body(x2_hbm, ww_hbm, wb_hbm, wf_hbm, out_hbm,
             idx0, idx1, rows0, rows1, acc_v, sem0, sem1):
        wid = lax.axis_index("s") * NC + lax.axis_index("c")
        tabs = (ww_hbm, wb_hbm, wf_hbm)
        idxs = (idx0, idx1)
        rows = (rows0, rows1)
        sems = (sem0, sem1)

        def issue(c, slot):
            off0 = pl.multiple_of((wid * BPW + c * CB) * L, ROWS)
            for t in range(3):
                pltpu.sync_copy(
                    x2_hbm.at[pl.ds(t * (B * L) + off0, ROWS)],
                    idxs[slot].at[pl.ds(t * ROWS, ROWS)])
            for t in range(3):
                for j in range(NG):
                    o = t * ROWS + j * G
                    pltpu.async_copy(
                        tabs[t].at[idxs[slot].at[pl.ds(o, G)]],
                        rows[slot].at[pl.ds(o, G)],
                        sems[slot])

        def drain(slot):
            pltpu.make_async_copy(
                tabs[0].at[pl.ds(0, 3 * ROWS)], rows[slot], sems[slot]).wait()

        def accumulate(c, slot):
            rv = rows[slot]
            for b in range(CB):
                def acc_body(l, carry):
                    a0, a1, g0, g1, f0, f1 = carry
                    r = b * L + l
                    fmt = plsc.PackFormat.INTERLEAVED
                    we, wo = plsc.unpack(
                        plsc.bitcast(rv[r], jnp.bfloat16), format=fmt)
                    ge, go = plsc.unpack(
                        plsc.bitcast(rv[ROWS + r], jnp.bfloat16), format=fmt)
                    fe, fo = plsc.unpack(
                        plsc.bitcast(rv[2 * ROWS + r], jnp.bfloat16),
                        format=fmt)
                    return (a0 + we, a1 + wo, g0 + ge, g1 + go,
                            f0 + fe, f1 + fo)
                z = jnp.zeros((LANES,), jnp.float32)
                a0, a1, g0, g1, f0, f1 = lax.fori_loop(
                    0, L, acc_body, (z, z, z, z, z, z))
                inv = jnp.float32(1.0 / L)
                o = (c * CB + b) * 2 * EMBED
                acc_v[pl.ds(o, LANES)] = (a0 + 5.0 * f0) * inv
                acc_v[pl.ds(o + LANES, LANES)] = (a1 + 5.0 * f1) * inv
                acc_v[pl.ds(o + 2 * LANES, LANES)] = g0 * inv
                acc_v[pl.ds(o + 3 * LANES, LANES)] = g1 * inv

        issue(0, 0)

        def pair_body(i, _):
            c0 = i * 2
            issue(c0 + 1, 1)
            drain(0)
            accumulate(c0, 0)

            @pl.when(c0 + 2 < NCHUNK)
            def _():
                issue(c0 + 2, 0)
            drain(1)
            accumulate(c0 + 1, 1)
            return 0

        lax.fori_loop(0, NCHUNK // 2, pair_body, 0)
        pltpu.sync_copy(
            acc_v, out_hbm.at[pl.ds(wid * BPW * 2 * EMBED, BPW * 2 * EMBED)])

    return body(x2, w_word, w_bigram, w_field)


VC = 1024
VQ = 125000
SS = 124928
MAIN = 8 * SS
TS = 72


_SEL_E = np.zeros((EMBED, EMBED // 2), np.float32)
_SEL_E[np.arange(0, EMBED, 2), np.arange(EMBED // 2)] = 1.0
_SEL_O = np.zeros((EMBED, EMBED // 2), np.float32)
_SEL_O[np.arange(1, EMBED, 2), np.arange(EMBED // 2)] = 1.0


def _pack_words(x, se, so):
    dn = (((0,), (0,)), ((), ()))
    e = lax.dot_general(x, se, dn, preferred_element_type=jnp.float32)
    o = lax.dot_general(x, so, dn, preferred_element_type=jnp.float32)
    ue = lax.bitcast_convert_type(e.astype(jnp.bfloat16), jnp.uint16)
    uo = lax.bitcast_convert_type(o.astype(jnp.bfloat16), jnp.uint16)
    word = (uo.astype(jnp.uint32) << 16) | ue.astype(jnp.uint32)
    return lax.bitcast_convert_type(word, jnp.float32)


def _repack_body(se_ref, so_ref, *refs):
    se = se_ref[...]
    so = so_ref[...]
    ins, out = refs[:8], refs[8]
    for s in range(8):
        out[:, 16 * s:16 * (s + 1)] = _pack_words(ins[s][...], se, so)


def _repack(wt):
    grid = (SS // VC,)
    sel_spec = pl.BlockSpec((EMBED, EMBED // 2), lambda i: (0, 0))
    in_specs = [sel_spec] * 2
    for s in range(8):
        off = s * (SS // VC)
        in_specs.append(pl.BlockSpec(
            (EMBED, VC), lambda i, off=off: (0, off + i)))
    out_spec = pl.BlockSpec((VC, 128), lambda i: (i, 0))
    shp = jax.ShapeDtypeStruct((SS, 128), jnp.float32)
    return pl.pallas_call(
        _repack_body,
        grid=grid,
        in_specs=in_specs,
        out_specs=out_spec,
        out_shape=shp,
        compiler_params=pltpu.CompilerParams(
            fuse_transposed_lhs_in_matmul=True),
    )(jnp.asarray(_SEL_E), jnp.asarray(_SEL_O), *([wt] * 8))


def _repack_tail_body(se_ref, so_ref, i3, out):
    se = se_ref[...]
    so = so_ref[...]
    x3 = i3[...]
    for s in range(8):
        out[:, 16 * s:16 * (s + 1)] = _pack_words(x3[:, s, :], se, so)


def _repack_tail(wt):
    t3 = wt[:, MAIN:].reshape(EMBED, 8, TS)
    shp = jax.ShapeDtypeStruct((TS, 128), jnp.float32)
    return pl.pallas_call(
        _repack_tail_body,
        out_shape=shp,
    )(jnp.asarray(_SEL_E), jnp.asarray(_SEL_O), t3)


_CPR = 128
_NCH = SS // _CPR


def _sc_linearize(m, t):
    mesh = plsc.VectorSubcoreMesh(core_axis_name="c", subcore_axis_name="s")
    shp = jax.ShapeDtypeStruct((VQ, 128), jnp.float32)

    @functools.partial(
        pl.kernel,
        mesh=mesh,
        out_type=shp,
        scratch_types=[pltpu.VMEM((_CPR, 128), jnp.float32),
                       pltpu.VMEM((_CPR, 128), jnp.float32),
                       pltpu.SemaphoreType.DMA,
                       pltpu.SemaphoreType.DMA],
        compiler_params=pltpu.CompilerParams(
            use_tc_tiling_on_sc=False, needs_layout_passes=False),
    )
    def body(m_hbm, t_hbm, out, vb0, vb1, sem0, sem1):
        wid = lax.axis_index("s") * NC + lax.axis_index("c")
        vbs = (vb0, vb1)
        sems = (sem0, sem1)

        def start(k, slot):
            cid = k * NW + wid

            @pl.when(cid < _NCH)
            def _():
                r0 = pl.multiple_of(cid * _CPR, _CPR)
                pltpu.async_copy(m_hbm.at[pl.ds(r0, _CPR)], vbs[slot],
                                 sems[slot])

        def flush(k, slot):
            cid = k * NW + wid

            @pl.when(cid < _NCH)
            def _():
                r0 = pl.multiple_of(cid * _CPR, _CPR)
                pltpu.make_async_copy(
                    m_hbm.at[pl.ds(0, _CPR)], vbs[slot], sems[slot]).wait()
                pltpu.sync_copy(vbs[slot], out.at[pl.ds(r0, _CPR)])

        nit = (_NCH + NW - 1) // NW
        start(0, 0)

        def pair_body(i, _):
            k0 = 2 * i
            start(k0 + 1, 1)
            flush(k0, 0)
            start(k0 + 2, 0)
            flush(k0 + 1, 1)
            return 0

        lax.fori_loop(0, nit // 2, pair_body, 0)
        if nit % 2:
            flush(nit - 1, 0)

        @pl.when(wid == 0)
        def _():
            pltpu.sync_copy(t_hbm, vb0.at[pl.ds(0, TS)])
            pltpu.sync_copy(vb0.at[pl.ds(0, TS)], out.at[pl.ds(SS, TS)])

    return body(m, t)


def _mlp_body(p_ref, w1_ref, b1_ref, w2_ref, b2_ref, o_ref):
    h = jnp.dot(p_ref[...], w1_ref[...], preferred_element_type=jnp.float32)
    h = jnp.maximum(h + b1_ref[...], 0.0)
    o_ref[...] = (
        jnp.dot(h, w2_ref[...], preferred_element_type=jnp.float32)
        + b2_ref[...]
    )


_PERM = np.concatenate([
    np.arange(0, EMBED, 2), np.arange(1, EMBED, 2),
    EMBED + np.arange(0, EMBED, 2), EMBED + np.arange(1, EMBED, 2),
])


def _mlp(pooled, fc1_w, fc1_b, fc2_w, fc2_b):
    w1p = fc1_w[_PERM, :]
    w2p = jnp.pad(fc2_w, ((0, 0), (0, HID - CLASSES)))
    b2p = jnp.pad(fc2_b, (0, HID - CLASSES)).reshape(1, HID)
    out = pl.pallas_call(
        _mlp_body,
        out_shape=jax.ShapeDtypeStruct((B, HID), jnp.float32),
    )(pooled, w1p, fc1_b.reshape(1, HID), w2p, b2p)
    return out[:, :CLASSES]


def kernel(x, W_word, W_bigram, W_field, fc1_w, fc1_b, fc2_w, fc2_b):
    xi = x.astype(jnp.int32)
    xt = xi - MAIN
    xp = jnp.where(xi < MAIN,
                   8 * (xi % SS) + xi // SS,
                   MAIN + 8 * (xt % TS) + xt // TS)
    x2 = xp.reshape(3 * B * L)
    ww, wb, wf = [_sc_linearize(_repack(wt), _repack_tail(wt))
                  for wt in (W_word.T, W_bigram.T, W_field.T)]
    p16 = (VQ * 8, EMBED // 2)
    pooled = _sc_pool(x2, ww.reshape(p16), wb.reshape(p16),
                      wf.reshape(p16)).reshape(B, 2 * EMBED)
    return _mlp(pooled, fc1_w, fc1_b, fc2_w, fc2_b)
